# baseline (speedup 1.0000x reference)
import dataclasses
import functools

import jax
import jax.numpy as jnp
from jax import lax
from jax.experimental import pallas as pl
from jax.experimental.pallas import tpu as pltpu
from jax.experimental.pallas import tpu_sc as plsc

N_USERS = 5000
N_NODES = 10000
N_EDGES = 160000
HALF = 128
N_LAYERS = 3
NS = 16
CH = 128
NCHUNK = 80
NQ = NCHUNK // 4
EPW = NCHUNK * CH
EPAD = NS * EPW
ACC_ROWS = 10240
DRAIN = ACC_ROWS // NS


def _sc_body(emb_hbm, epk_hbm, vp_hbm, o1, o2, o3,
             ebuf, vbuf, bufA, bufB, acc,
             esem0, esem1, esem2, esem3, gsemA, gsemB, ssemA, ssemB):
    cid = lax.axis_index("c")
    sid = lax.axis_index("s")
    esems = [esem0, esem1, esem2, esem3]
    ep = epk_hbm.at[sid]
    vp = vp_hbm.at[sid]
    z16 = jnp.zeros((16,), jnp.float32)

    def load_idx(j, slot):
        pltpu.async_copy(ep.at[j], ebuf.at[slot], esems[slot])
        pltpu.async_copy(vp.at[j], vbuf.at[slot], esems[slot])

    def wait_idx(slot):
        pltpu.make_async_copy(ep.at[0], ebuf.at[slot], esems[slot]).wait()
        pltpu.make_async_copy(vp.at[0], vbuf.at[slot], esems[slot]).wait()

    def scale(buf, slot):
        sv = jnp.full((16,), slot, jnp.int32)

        @pl.loop(0, CH)
        def _(e):
            ev = jnp.full((16,), e, jnp.int32)
            v = plsc.load_gather(vbuf, [sv, ev])
            for g in range(8):
                sl = pl.ds(g * 16, 16)
                buf[e, sl] = buf[e, sl] * v

    def do_layer(src, dst):
        srcc = src.at[cid]

        @pl.loop(0, CH)
        def _(r):
            for g in range(8):
                bufA[r, pl.ds(g * 16, 16)] = z16

        for z in range(5):
            pltpu.sync_copy(bufA, acc.at[pl.ds(sid * DRAIN + z * CH, CH)])
        plsc.subcore_barrier()

        load_idx(0, 0)
        load_idx(1, 1)
        wait_idx(0)
        pltpu.async_copy(srcc.at[ebuf.at[0, 0]], bufA, gsemA)

        def item(q, i):
            j = 4 * q + i
            if i % 2 == 0:
                bme, gme, sme = bufA, gsemA, ssemA
                bot, got, sot = bufB, gsemB, ssemB
            else:
                bme, gme, sme = bufB, gsemB, ssemB
                bot, got, sot = bufA, gsemA, ssemA
            s_nxt = (i + 1) % 4
            s_pre = (i + 2) % 4

            pltpu.make_async_copy(srcc.at[ebuf.at[i, 0]], bme, gme).wait()
            scale(bme, i)

            if i == 0:
                @pl.when(q > 0)
                def _():
                    pltpu.make_async_copy(bot, acc.at[ebuf.at[3, 1]], sot).wait()
            else:
                pltpu.make_async_copy(bot, acc.at[ebuf.at[i - 1, 1]], sot).wait()

            @pl.when(j + 2 < NCHUNK)
            def _():
                load_idx(j + 2, s_pre)

            @pl.when(j + 1 < NCHUNK)
            def _():
                wait_idx(s_nxt)
                pltpu.async_copy(srcc.at[ebuf.at[s_nxt, 0]], bot, got)

            pltpu.async_copy(bme, acc.at[ebuf.at[i, 1]], sme, add=True)

        @pl.loop(0, NQ)
        def _(q):
            for i in range(4):
                item(q, i)

        pltpu.make_async_copy(bufB, acc.at[ebuf.at[3, 1]], ssemB).wait()
        plsc.subcore_barrier()
        pltpu.sync_copy(acc.at[pl.ds(sid * DRAIN, DRAIN)],
                        dst.at[cid].at[pl.ds(sid * DRAIN, DRAIN)])
        plsc.subcore_barrier()

    do_layer(emb_hbm, o1)
    do_layer(o1, o2)
    do_layer(o2, o3)


def _combine_body(e_ref, a_ref, b_ref, c_ref, o_ref):
    o_ref[...] = (e_ref[...] + a_ref[...] + b_ref[...] + c_ref[...]) * 0.25


def kernel(user_embedding, item_embedding, adj_indices, adj_values):
    f32 = jnp.float32
    all_emb = jnp.concatenate([user_embedding, item_embedding], axis=0)
    emb2 = all_emb.reshape(N_NODES, 2, HALF).transpose(1, 0, 2)
    emb2 = jnp.pad(emb2, ((0, 0), (0, ACC_ROWS - N_NODES), (0, 0)))

    pad = EPAD - N_EDGES
    row = jnp.concatenate([adj_indices[0], jnp.zeros((pad,), jnp.int32)])
    col = jnp.concatenate([adj_indices[1], jnp.zeros((pad,), jnp.int32)])
    val = jnp.concatenate([adj_values, jnp.zeros((pad,), f32)])
    packed = jnp.stack(
        [col.reshape(NS, NCHUNK, CH),
         row.reshape(NS, NCHUNK, CH)], axis=2)
    valp = val.reshape(NS, NCHUNK, CH)

    cp = pltpu.CompilerParams()
    if "needs_layout_passes" in pltpu.CompilerParams.__dataclass_fields__:
        cp = dataclasses.replace(cp, needs_layout_passes=False)

    out_sds = jax.ShapeDtypeStruct((2, ACC_ROWS, HALF), f32)
    sc_fn = pl.kernel(
        _sc_body,
        out_type=[out_sds, out_sds, out_sds],
        mesh=plsc.VectorSubcoreMesh(core_axis_name="c", subcore_axis_name="s"),
        scratch_types=[
            pltpu.VMEM((4, 2, CH), jnp.int32),
            pltpu.VMEM((4, CH), f32),
            pltpu.VMEM((CH, HALF), f32),
            pltpu.VMEM((CH, HALF), f32),
            pltpu.VMEM_SHARED((ACC_ROWS, HALF), f32),
            pltpu.SemaphoreType.DMA,
            pltpu.SemaphoreType.DMA,
            pltpu.SemaphoreType.DMA,
            pltpu.SemaphoreType.DMA,
            pltpu.SemaphoreType.DMA,
            pltpu.SemaphoreType.DMA,
            pltpu.SemaphoreType.DMA,
            pltpu.SemaphoreType.DMA,
        ],
        compiler_params=cp,
    )
    l1, l2, l3 = sc_fn(emb2, packed, valp)

    combined = pl.pallas_call(
        _combine_body,
        out_shape=jax.ShapeDtypeStruct((2, ACC_ROWS, HALF), f32),
        grid=(2, 10),
        in_specs=[pl.BlockSpec((1, ACC_ROWS // 10, HALF),
                               lambda i, j: (i, j, 0))] * 4,
        out_specs=pl.BlockSpec((1, ACC_ROWS // 10, HALF), lambda i, j: (i, j, 0)),
    )(emb2, l1, l2, l3)

    final = combined[:, :N_NODES, :].transpose(1, 0, 2).reshape(N_NODES, 2 * HALF)
    return (final[:N_USERS], final[N_USERS:])
